# Initial kernel scaffold; baseline (speedup 1.0000x reference)
import functools

import jax
import jax.numpy as jnp
from jax import lax
from jax.experimental import pallas as pl
from jax.experimental.pallas import tpu as pltpu
from jax.experimental.pallas import tpu_sc as plsc

V = 100000
D = 64
B = 16384
H = 50
N = B * H
DC = 16
DO = D + 2 * DC

NC, NS = 2, 16
NW = NC * NS
PER_W = N // NW
G = 128
K = 4
C = K * G
N_CHUNKS = PER_W // C


def _combine_tables(symbol_embed, layer_embeds, layer):
    rows_blk = 1000
    grid = V // rows_blk

    def body(layer_ref, sym_ref, lay_ref, out_ref):
        del layer_ref
        out_ref[...] = sym_ref[...] + lay_ref[0]

    return pl.pallas_call(
        body,
        grid_spec=pltpu.PrefetchScalarGridSpec(
            num_scalar_prefetch=1,
            grid=(grid,),
            in_specs=[
                pl.BlockSpec((rows_blk, D), lambda i, layer_ref: (i, 0)),
                pl.BlockSpec((1, rows_blk, D),
                             lambda i, layer_ref: (layer_ref[0], i, 0)),
            ],
            out_specs=pl.BlockSpec((rows_blk, D), lambda i, layer_ref: (i, 0)),
        ),
        out_shape=jax.ShapeDtypeStruct((V, D), jnp.float32),
    )(jnp.asarray(layer, jnp.int32).reshape(1), symbol_embed, layer_embeds)


def _sc_lookup(comb, cal_table, cos_table, sids2, cal2, cos2):
    mesh = plsc.VectorSubcoreMesh(core_axis_name="c", subcore_axis_name="s")

    @functools.partial(
        pl.kernel,
        out_type=jax.ShapeDtypeStruct((N, DO), jnp.float32),
        mesh=mesh,
        scratch_types=[
            pltpu.VMEM((K, G), jnp.int32),
            pltpu.VMEM((K, G), jnp.int32),
            pltpu.VMEM((K, G), jnp.int32),
            pltpu.VMEM((C, D), jnp.float32),
            pltpu.VMEM((C, DC), jnp.float32),
            pltpu.VMEM((C, DC), jnp.float32),
            pltpu.SemaphoreType.DMA,
        ],
    )
    def kern(comb_hbm, calt_hbm, cost_hbm, sid_hbm, cal_hbm, cos_hbm,
             out_hbm, idx_s, idx_c, idx_k, rows, calr, cosr, sem):
        wid = lax.axis_index("s") * NC + lax.axis_index("c")

        def chunk(i, carry):
            base = wid * PER_W + i * C
            rb = wid * (PER_W // G) + i * K
            pltpu.sync_copy(sid_hbm.at[pl.ds(rb, K)], idx_s)
            pltpu.sync_copy(cal_hbm.at[pl.ds(rb, K)], idx_c)
            pltpu.sync_copy(cos_hbm.at[pl.ds(rb, K)], idx_k)
            copies = []
            for j in range(K):
                copies.append(pltpu.async_copy(
                    comb_hbm.at[idx_s.at[j]],
                    rows.at[pl.ds(j * G, G)], sem))
                copies.append(pltpu.async_copy(
                    calt_hbm.at[idx_c.at[j]],
                    calr.at[pl.ds(j * G, G)], sem))
                copies.append(pltpu.async_copy(
                    cost_hbm.at[idx_k.at[j]],
                    cosr.at[pl.ds(j * G, G)], sem))
            for cp in copies:
                cp.wait()
            pltpu.sync_copy(rows, out_hbm.at[pl.ds(base, C), pl.ds(0, D)])
            pltpu.sync_copy(calr, out_hbm.at[pl.ds(base, C), pl.ds(D, DC)])
            pltpu.sync_copy(cosr, out_hbm.at[pl.ds(base, C), pl.ds(D + DC, DC)])
            return carry

        lax.fori_loop(0, N_CHUNKS, chunk, 0)

    return kern(comb, cal_table, cos_table, sids2, cal2, cos2)


def kernel(symbol_ids, layer, calendar_pos, cosmic_assoc, symbol_embed,
           layer_embeds, calendar_embed, cosmic_embed):
    comb = _combine_tables(symbol_embed, layer_embeds, layer)
    sids2 = symbol_ids.astype(jnp.int32).reshape(N // G, G)
    cal2 = calendar_pos.astype(jnp.int32).reshape(N // G, G)
    cos2 = cosmic_assoc.astype(jnp.int32).reshape(N // G, G)
    out = _sc_lookup(comb, calendar_embed, cosmic_embed, sids2, cal2, cos2)
    return out.reshape(B, H, DO)


# speedup vs baseline: 5.7812x; 5.7812x over previous
import functools

import jax
import jax.numpy as jnp
from jax import lax
from jax.experimental import pallas as pl
from jax.experimental.pallas import tpu as pltpu
from jax.experimental.pallas import tpu_sc as plsc

V = 100000
D = 64
DP = 128
B = 16384
H = 50
N = B * H
DC = 16
DO = D + 2 * DC
NCAL = 260
NCOS = 20

NC, NS = 2, 16
NW = NC * NS
PER_W = N // NW
G = 128
K = 2
C = K * G
N_CHUNKS = PER_W // C


def _combine_tables(symbol_embed, layer_embeds, layer):
    rows_blk = 1000
    grid = V // rows_blk

    def body(layer_ref, sym_ref, lay_ref, out_ref):
        del layer_ref
        s = sym_ref[...] + lay_ref[0]
        out_ref[...] = jnp.concatenate(
            [s, jnp.zeros((rows_blk, DP - D), jnp.float32)], axis=-1)

    return pl.pallas_call(
        body,
        grid_spec=pltpu.PrefetchScalarGridSpec(
            num_scalar_prefetch=1,
            grid=(grid,),
            in_specs=[
                pl.BlockSpec((rows_blk, D), lambda i, layer_ref: (i, 0)),
                pl.BlockSpec((1, rows_blk, D),
                             lambda i, layer_ref: (layer_ref[0], i, 0)),
            ],
            out_specs=pl.BlockSpec((rows_blk, DP), lambda i, layer_ref: (i, 0)),
        ),
        out_shape=jax.ShapeDtypeStruct((V, DP), jnp.float32),
    )(jnp.asarray(layer, jnp.int32).reshape(1), symbol_embed, layer_embeds)


def _sc_lookup(comb, cal_flat, cos_flat, sids, cal, cos):
    mesh = plsc.VectorSubcoreMesh(core_axis_name="c", subcore_axis_name="s")

    @functools.partial(
        pl.kernel,
        out_type=jax.ShapeDtypeStruct((N, DO), jnp.float32),
        mesh=mesh,
        scratch_types=[
            pltpu.VMEM((C,), jnp.int32),
            pltpu.VMEM((C,), jnp.int32),
            pltpu.VMEM((C,), jnp.int32),
            pltpu.VMEM((C, DP), jnp.float32),
            pltpu.VMEM((C, DO), jnp.float32),
            pltpu.VMEM((NCAL * DC,), jnp.float32),
            pltpu.VMEM((NCOS * DC,), jnp.float32),
            pltpu.SemaphoreType.DMA,
        ],
        compiler_params=pltpu.CompilerParams(needs_layout_passes=False),
    )
    def kern(comb_hbm, calt_hbm, cost_hbm, sid_hbm, cal_hbm, cos_hbm,
             out_hbm, idx_s, idx_c, idx_k, rows, obuf, calv, cosv, sem):
        wid = lax.axis_index("s") * NC + lax.axis_index("c")
        pltpu.sync_copy(calt_hbm, calv)
        pltpu.sync_copy(cost_hbm, cosv)
        iota16 = lax.iota(jnp.int32, 16)

        def chunk(i, carry):
            base = wid * PER_W + i * C
            pltpu.sync_copy(sid_hbm.at[pl.ds(base, C)], idx_s)
            pltpu.sync_copy(cal_hbm.at[pl.ds(base, C)], idx_c)
            pltpu.sync_copy(cos_hbm.at[pl.ds(base, C)], idx_k)
            copies = []
            for j in range(K):
                copies.append(pltpu.async_copy(
                    comb_hbm.at[idx_s.at[pl.ds(j * G, G)]],
                    rows.at[pl.ds(j * G, G)], sem))
            for cp in copies:
                cp.wait()

            def group(g, carry2):
                rb = g * 16
                vc16 = idx_c[pl.ds(rb, 16)] * DC
                vk16 = idx_k[pl.ds(rb, 16)] * DC
                rowv = rb + iota16
                for c in range(DC):
                    w = plsc.load_gather(calv, [vc16 + c])
                    plsc.store_scatter(
                        obuf, [rowv, jnp.full((16,), D + c, jnp.int32)], w)
                    w = plsc.load_gather(cosv, [vk16 + c])
                    plsc.store_scatter(
                        obuf, [rowv, jnp.full((16,), D + DC + c, jnp.int32)], w)
                for l in range(16):
                    for c in range(D // 16):
                        obuf[rb + l, pl.ds(c * 16, 16)] = (
                            rows[rb + l, pl.ds(c * 16, 16)])
                return carry2

            lax.fori_loop(0, C // 16, group, 0)
            pltpu.sync_copy(obuf, out_hbm.at[pl.ds(base, C)])
            return carry

        lax.fori_loop(0, N_CHUNKS, chunk, 0)

    return kern(comb, cal_flat, cos_flat, sids, cal, cos)


def kernel(symbol_ids, layer, calendar_pos, cosmic_assoc, symbol_embed,
           layer_embeds, calendar_embed, cosmic_embed):
    comb = _combine_tables(symbol_embed, layer_embeds, layer)
    sids = symbol_ids.astype(jnp.int32).reshape(N)
    cal = calendar_pos.astype(jnp.int32).reshape(N)
    cos = cosmic_assoc.astype(jnp.int32).reshape(N)
    out = _sc_lookup(comb, calendar_embed.reshape(NCAL * DC),
                     cosmic_embed.reshape(NCOS * DC), sids, cal, cos)
    return out.reshape(B, H, DO)
